# baseline (speedup 1.0000x reference)
import functools

import jax
import jax.numpy as jnp
from jax import lax
from jax.experimental import pallas as pl
from jax.experimental.pallas import tpu as pltpu
from jax.experimental.pallas import tpu_sc as plsc

NA = 10000
NB = 10000
NE = 160000
IN_DIM = 256
HID = 512
NH = 4
HD = 128
NCLS = 40

NAP = 10240
NSUB = 16
EPT = NE // NSUB
RPT = 624
CE = 2000
KG = 16
ZR = 16
HW = 128
NSPLIT = HD // HW
NHH = NH * NSPLIT
DR = NAP // 128


BN1 = 400


def _proj_body(x_ref, w_ref, b_ref, watt_ref, *out_refs):
    ka = watt_ref.shape[1]
    ht_refs = out_refs[:NHH]
    av_refs = out_refs[NHH:]
    h = jnp.dot(x_ref[...], w_ref[...], preferred_element_type=jnp.float32)
    h = h + b_ref[...]
    for j in range(NHH):
        ht_refs[j][...] = h[:, j * HW:(j + 1) * HW]
    av = jnp.dot(h, watt_ref[...], preferred_element_type=jnp.float32)
    for j in range(ka):
        av_refs[j][...] = av[:, j:j + 1]


def _project(x, w, b, watt):
    n = x.shape[0]
    ka = watt.shape[1]
    grid = (n // BN1,)
    return pl.pallas_call(
        _proj_body,
        grid=grid,
        in_specs=[
            pl.BlockSpec((BN1, IN_DIM), lambda i: (i, 0)),
            pl.BlockSpec((IN_DIM, HID), lambda i: (0, 0)),
            pl.BlockSpec((1, HID), lambda i: (0, 0)),
            pl.BlockSpec((HID, ka), lambda i: (0, 0)),
        ],
        out_specs=(
            [pl.BlockSpec((BN1, HW), lambda i: (i, 0)) for _ in range(NHH)]
            + [pl.BlockSpec((BN1, 1), lambda i: (i, 0)) for _ in range(ka)]
        ),
        out_shape=(
            [jax.ShapeDtypeStruct((n, HW), jnp.float32) for _ in range(NHH)]
            + [jax.ShapeDtypeStruct((n, 1), jnp.float32) for _ in range(ka)]
        ),
    )(x, w, b, watt)


def _sc_edge_conv_body(
    src0, dst0, as0, ad0, ht0,
    src1, dst1, as1, ad1, ht1,
    out_hbm, ex_hbm,
    dloc, abuf, srcb, dstb, rowbuf, gidx, sidx, idx80,
    den_sp, out_sp, sem, ssem,
):
    cid = lax.axis_index("c")
    sid = lax.axis_index("s")
    zero16 = jnp.zeros((16,), jnp.float32)
    lanes = lax.iota(jnp.int32, 16)

    def zero_rowbuf():
        def zrow(r, carry):
            for v in range(HW // 16):
                rowbuf[r, pl.ds(16 * v, 16)] = zero16
            return carry

        lax.fori_loop(0, KG, zrow, 0)

    def zero_dloc():
        def zrow(r, carry):
            for v in range(8):
                dloc[r, pl.ds(16 * v, 16)] = zero16
            return carry

        lax.fori_loop(0, DR, zrow, 0)

    def run(t, src_h, dst_h, as_h, ad_h, ht_h):
        ebase = sid * EPT
        row_start = sid * RPT
        nchunks = jnp.where(sid == NSUB - 1, (NA - RPT * (NSUB - 1)) // ZR,
                            RPT // ZR)

        zero_dloc()

        @pl.when(sid == 0)
        def _():
            for h in range(NH):
                pltpu.sync_copy(dloc, den_sp[h])

        for u in range(DR // 16):
            idx80[pl.ds(16 * u, 16)] = lanes + (16 * u)
        plsc.subcore_barrier()

        def phase1(ast, adt):
          for h in range(NH):
            pltpu.sync_copy(as_h[h], ast)
            pltpu.sync_copy(ad_h[h], adt)
            zero_dloc()

            def p1chunk(c, carry, h=h):
                base = pl.multiple_of(ebase + c * CE, 8)
                pltpu.sync_copy(src_h.at[pl.ds(base, CE)], srcb)
                pltpu.sync_copy(dst_h.at[pl.ds(base, CE)], dstb)

                def p1body(j, carry2):
                    s16 = srcb[pl.ds(j * 16, 16)]
                    d16 = dstb[pl.ds(j * 16, 16)]
                    av = plsc.load_gather(ast, [s16])
                    bv = plsc.load_gather(adt, [d16])
                    al = av + bv
                    al = jnp.where(al >= 0.0, al, al * 0.2)
                    ex = jnp.exp(al)
                    abuf[pl.ds(j * 16, 16)] = ex
                    plsc.addupdate_scatter(
                        dloc, [lax.shift_right_logical(d16, 7),
                               jnp.bitwise_and(d16, 127)], ex)
                    return carry2

                lax.fori_loop(0, CE // 16, p1body, 0)
                pltpu.sync_copy(abuf, ex_hbm[t * NH + h].at[pl.ds(base, CE)])
                return carry

            lax.fori_loop(0, EPT // CE, p1chunk, 0)
            pltpu.sync_copy(dloc, den_sp[h].at[idx80], add=True)

        pl.run_scoped(phase1,
                      pltpu.VMEM((NA,), jnp.float32),
                      pltpu.VMEM((NA,), jnp.float32))
        plsc.subcore_barrier()

        def phase2(msgbuf):
          for h in range(NH):
            pltpu.sync_copy(den_sp[h], dloc)
            zero_rowbuf()

            def zcopy(j, carry):
                off = pl.multiple_of(row_start + j * ZR, ZR)
                pltpu.sync_copy(rowbuf.at[pl.ds(0, ZR)],
                                out_sp.at[pl.ds(off, ZR)])
                return carry

            lax.fori_loop(0, nchunks, zcopy, 0)
            plsc.subcore_barrier()

            def p2chunk(c, carry, h=h):
                base = pl.multiple_of(ebase + c * CE, 8)
                pltpu.sync_copy(src_h.at[pl.ds(base, CE)], srcb)
                pltpu.sync_copy(dst_h.at[pl.ds(base, CE)], dstb)
                pltpu.sync_copy(ex_hbm[t * NH + h].at[pl.ds(base, CE)], abuf)

                def nrm(j, carry2):
                    d16 = dstb[pl.ds(j * 16, 16)]
                    den = plsc.load_gather(
                        dloc, [lax.shift_right_logical(d16, 7),
                               jnp.bitwise_and(d16, 127)])
                    ex = abuf[pl.ds(j * 16, 16)]
                    abuf[pl.ds(j * 16, 16)] = ex / (den + 1e-16)
                    return carry2

                lax.fori_loop(0, CE // 16, nrm, 0)

                def sub(g, carry2, h=h):
                    @pl.when(g > 0)
                    def _():
                        pltpu.make_async_copy(msgbuf, out_sp.at[sidx],
                                              ssem).wait()

                    for u in range(KG // 16):
                        gidx[pl.ds(16 * u, 16)] = srcb[pl.ds(g * KG + 16 * u, 16)]
                        sidx[pl.ds(16 * u, 16)] = dstb[pl.ds(g * KG + 16 * u, 16)]
                    pltpu.async_copy(ht_h[h].at[gidx], rowbuf, sem).wait()

                    def scale(qg, carry3):
                        av16 = abuf[pl.ds(g * KG + qg * 16, 16)]
                        for i in range(16):
                            a = av16[i]
                            r = qg * 16 + i
                            for v in range(HW // 16):
                                msgbuf[r, pl.ds(16 * v, 16)] = (
                                    rowbuf[r, pl.ds(16 * v, 16)] * a
                                )
                        return carry3

                    lax.fori_loop(0, KG // 16, scale, 0)
                    pltpu.async_copy(msgbuf, out_sp.at[sidx], ssem, add=True)
                    return carry2

                lax.fori_loop(0, CE // KG, sub, 0)
                pltpu.make_async_copy(msgbuf, out_sp.at[sidx], ssem).wait()
                return carry

            lax.fori_loop(0, EPT // CE, p2chunk, 0)
            plsc.subcore_barrier()

            def dcopy(j, carry, h=h):
                off = pl.multiple_of(row_start + j * ZR, ZR)
                pltpu.sync_copy(out_sp.at[pl.ds(off, ZR)],
                                out_hbm.at[t, h, pl.ds(off, ZR)])
                return carry

            lax.fori_loop(0, nchunks, dcopy, 0)
            plsc.subcore_barrier()

        pl.run_scoped(phase2, pltpu.VMEM((KG, HW), jnp.float32))

    @pl.when(cid == 0)
    def _():
        run(0, src0, dst0, as0, ad0, ht0)

    @pl.when(cid == 1)
    def _():
        run(1, src1, dst1, as1, ad1, ht1)


def _sc_edge_conv(src0, dst0, as0, ad0, ht0, src1, dst1, as1, ad1, ht1):
    mesh = plsc.VectorSubcoreMesh(core_axis_name="c", subcore_axis_name="s",
                                  num_cores=2)
    f = pl.kernel(
        _sc_edge_conv_body,
        out_type=(
            jax.ShapeDtypeStruct((2, NHH, NA, HW), jnp.float32),
            [jax.ShapeDtypeStruct((NE,), jnp.float32) for _ in range(2 * NH)],
        ),
        mesh=mesh,
        compiler_params=pltpu.CompilerParams(needs_layout_passes=False),
        scratch_types=[
            pltpu.VMEM((DR, 128), jnp.float32),
            pltpu.VMEM((CE,), jnp.float32),
            pltpu.VMEM((CE,), jnp.int32),
            pltpu.VMEM((CE,), jnp.int32),
            pltpu.VMEM((KG, HW), jnp.float32),
            pltpu.VMEM((KG,), jnp.int32),
            pltpu.VMEM((KG,), jnp.int32),
            pltpu.VMEM((DR,), jnp.int32),
            [pltpu.VMEM_SHARED((DR, 128), jnp.float32) for _ in range(NH)],
            pltpu.VMEM_SHARED((NA, HW), jnp.float32),
            pltpu.SemaphoreType.DMA,
            pltpu.SemaphoreType.DMA,
        ],
    )
    out, _ex = f(src0, dst0, as0, ad0, ht0, src1, dst1, as1, ad1, ht1)
    return out


BN2 = 1000


def _sem_body(o_ref, kw_ref, kb_ref, s_ref):
    i = pl.program_id(1)
    blk = jnp.concatenate([o_ref[0, j] for j in range(NHH)], axis=-1)
    blk = jnp.maximum(blk, 0.0)
    kk = jnp.tanh(
        jnp.dot(blk, kw_ref[...], preferred_element_type=jnp.float32)
        + kb_ref[...]
    )
    part = jnp.sum(kk, axis=0, keepdims=True)[None]

    @pl.when(i == 0)
    def _():
        s_ref[...] = jnp.zeros_like(s_ref)

    s_ref[...] += part


def _sem_sums(out_sc, k_w, k_b):
    return pl.pallas_call(
        _sem_body,
        grid=(2, NA // BN2),
        in_specs=[
            pl.BlockSpec((1, NHH, BN2, HW), lambda t, i: (t, 0, i, 0)),
            pl.BlockSpec((HID, HID), lambda t, i: (0, 0)),
            pl.BlockSpec((1, HID), lambda t, i: (0, 0)),
        ],
        out_specs=pl.BlockSpec((1, 1, HID), lambda t, i: (t, 0, 0)),
        out_shape=jax.ShapeDtypeStruct((2, 1, HID), jnp.float32),
    )(out_sc, k_w, k_b)


def _fuse_body(o_ref, s_ref, q_ref, lw_ref, lb_ref, logits_ref, hrep_ref):
    qv = q_ref[...]
    s0 = jnp.sum(qv[0] * s_ref[0, 0]) / NA
    s1 = jnp.sum(qv[0] * s_ref[1, 0]) / NA
    m = jnp.maximum(s0, s1)
    e0 = jnp.exp(s0 - m)
    e1 = jnp.exp(s1 - m)
    den = e0 + e1
    a0 = e0 / den
    a1 = e1 / den
    b0 = jnp.concatenate([o_ref[0, j] for j in range(NHH)], axis=-1)
    b1 = jnp.concatenate([o_ref[1, j] for j in range(NHH)], axis=-1)
    b0 = jnp.maximum(b0, 0.0)
    b1 = jnp.maximum(b1, 0.0)
    fused = a0 * b0 + a1 * b1
    logits_ref[...] = (
        jnp.dot(fused, lw_ref[...], preferred_element_type=jnp.float32)
        + lb_ref[...]
    )
    hrep_ref[...] = jnp.broadcast_to(fused[:, None, :], (BN2, 2, HID))


def _fuse(out_sc, s2, q, lin_w, lin_b):
    return pl.pallas_call(
        _fuse_body,
        grid=(NA // BN2,),
        in_specs=[
            pl.BlockSpec((2, NHH, BN2, HW), lambda i: (0, 0, i, 0)),
            pl.BlockSpec((2, 1, HID), lambda i: (0, 0, 0)),
            pl.BlockSpec((1, HID), lambda i: (0, 0)),
            pl.BlockSpec((HID, NCLS), lambda i: (0, 0)),
            pl.BlockSpec((1, NCLS), lambda i: (0, 0)),
        ],
        out_specs=[
            pl.BlockSpec((BN2, NCLS), lambda i: (i, 0)),
            pl.BlockSpec((BN2, 2, HID), lambda i: (i, 0, 0)),
        ],
        out_shape=[
            jax.ShapeDtypeStruct((NA, NCLS), jnp.float32),
            jax.ShapeDtypeStruct((NA, 2, HID), jnp.float32),
        ],
    )(out_sc, s2, q, lin_w, lin_b)


def _att_matrix(att_stack):
    att_t = jnp.transpose(att_stack, (1, 2, 0))
    w = att_t[:, :, :, None] * jnp.eye(NH, dtype=att_stack.dtype)[:, None, None, :]
    return w.reshape(HID, att_stack.shape[0] * NH)


def kernel(x_A, x_B, edge_index_ba, edge_index_aa, W_A, b_A, W_B, b_B,
           att_src_ba, att_dst_ba, att_src_aa, att_dst_aa, k_W, k_b, q,
           lin_W, lin_b):
    att_a = jnp.stack(
        [att_dst_ba[0], att_src_aa[0], att_dst_aa[0]]
    )
    att_b = att_src_ba
    watt_a = _att_matrix(att_a)
    watt_b = _att_matrix(att_b)

    outs_a = _project(x_A, W_A, b_A.reshape(1, HID), watt_a)
    outs_b = _project(x_B, W_B, b_B.reshape(1, HID), watt_b)
    htA = tuple(outs_a[:NHH])
    htB = tuple(outs_b[:NHH])
    avA = [a.reshape(NA) for a in outs_a[NHH:]]
    avB = [a.reshape(NB) for a in outs_b[NHH:]]
    ad_ba = tuple(avA[0:NH])
    as_aa = tuple(avA[NH:2 * NH])
    ad_aa = tuple(avA[2 * NH:3 * NH])
    as_ba = tuple(avB)

    out_sc = _sc_edge_conv(
        edge_index_ba[0], edge_index_ba[1], as_ba, ad_ba, htB,
        edge_index_aa[0], edge_index_aa[1], as_aa, ad_aa, htA,
    )

    s2 = _sem_sums(out_sc, k_W, k_b.reshape(1, HID))
    logits, hrep = _fuse(out_sc, s2, q.reshape(1, HID), lin_W,
                         lin_b.reshape(1, NCLS))

    alpha = jnp.full((NA, 2), 0.5, jnp.float32)
    return (logits, hrep, alpha)


# speedup vs baseline: 1.6706x; 1.6706x over previous
import functools

import jax
import jax.numpy as jnp
from jax import lax
from jax.experimental import pallas as pl
from jax.experimental.pallas import tpu as pltpu
from jax.experimental.pallas import tpu_sc as plsc

NA = 10000
NB = 10000
NE = 160000
IN_DIM = 256
HID = 512
NH = 4
HD = 128
NCLS = 40

NAP = 10240
NSUB = 16
EPT = NE // NSUB
RPT = 624
CE = 2000
KG = 80
ZR = 16
HW = 128
NSPLIT = HD // HW
NHH = NH * NSPLIT
DR = NAP // 128


BN1 = 512


def _proj_body(x_ref, w_ref, b_ref, watt_ref, *out_refs):
    ka = watt_ref.shape[1]
    ht_refs = out_refs[:NHH]
    av_refs = out_refs[NHH:]
    h = jnp.dot(x_ref[...], w_ref[...], preferred_element_type=jnp.float32)
    h = h + b_ref[...]
    for j in range(NHH):
        ht_refs[j][...] = h[:, j * HW:(j + 1) * HW]
    av = jnp.dot(h, watt_ref[...], preferred_element_type=jnp.float32)
    for j in range(ka):
        av_refs[j][...] = av[:, j:j + 1]


def _project(x, w, b, watt):
    n = x.shape[0]
    ka = watt.shape[1]
    grid = (n // BN1,)
    return pl.pallas_call(
        _proj_body,
        grid=grid,
        in_specs=[
            pl.BlockSpec((BN1, IN_DIM), lambda i: (i, 0)),
            pl.BlockSpec((IN_DIM, HID), lambda i: (0, 0)),
            pl.BlockSpec((1, HID), lambda i: (0, 0)),
            pl.BlockSpec((HID, ka), lambda i: (0, 0)),
        ],
        out_specs=(
            [pl.BlockSpec((BN1, HW), lambda i: (i, 0)) for _ in range(NHH)]
            + [pl.BlockSpec((BN1, 1), lambda i: (i, 0)) for _ in range(ka)]
        ),
        out_shape=(
            [jax.ShapeDtypeStruct((n, HW), jnp.float32) for _ in range(NHH)]
            + [jax.ShapeDtypeStruct((n, 1), jnp.float32) for _ in range(ka)]
        ),
    )(x, w, b, watt)


def _sc_edge_conv_body(
    src0, dst0, as0, ad0, ht0,
    src1, dst1, as1, ad1, ht1,
    out_hbm, ex_hbm,
    bufa, bufb, dloc, abuf, srcb, dstb, gidx, sidx, idx80,
    den_sp, out_sp, sem, ssem,
):
    cid = lax.axis_index("c")
    sid = lax.axis_index("s")
    zero16 = jnp.zeros((16,), jnp.float32)
    lanes = lax.iota(jnp.int32, 16)

    def zero_buf(buf):
        def zrow(r, carry):
            for v in range(HW // 16):
                buf[r, pl.ds(16 * v, 16)] = zero16
            return carry

        lax.fori_loop(0, KG, zrow, 0)

    def zero_dloc():
        def zrow(r, carry):
            for v in range(8):
                dloc[r, pl.ds(16 * v, 16)] = zero16
            return carry

        lax.fori_loop(0, DR, zrow, 0)

    def run(t, src_h, dst_h, as_h, ad_h, ht_h):
        ebase = sid * EPT
        row_start = sid * RPT
        nchunks = jnp.where(sid == NSUB - 1, (NA - RPT * (NSUB - 1)) // ZR,
                            RPT // ZR)

        zero_dloc()

        @pl.when(sid == 0)
        def _():
            for h in range(NH):
                pltpu.sync_copy(dloc, den_sp[h])

        for u in range(DR // 16):
            idx80[pl.ds(16 * u, 16)] = lanes + (16 * u)
        plsc.subcore_barrier()

        for h in range(NH):
            pltpu.sync_copy(as_h[h], bufa)
            pltpu.sync_copy(ad_h[h], bufb)
            zero_dloc()

            def p1chunk(c, carry, h=h):
                base = pl.multiple_of(ebase + c * CE, 8)
                pltpu.sync_copy(src_h.at[pl.ds(base, CE)], srcb)
                pltpu.sync_copy(dst_h.at[pl.ds(base, CE)], dstb)

                def p1body(j, carry2):
                    s16 = srcb[pl.ds(j * 16, 16)]
                    d16 = dstb[pl.ds(j * 16, 16)]
                    sr = lax.shift_right_logical(s16, 7)
                    sc = jnp.bitwise_and(s16, 127)
                    dr = lax.shift_right_logical(d16, 7)
                    dc = jnp.bitwise_and(d16, 127)
                    av = plsc.load_gather(bufa, [sr, sc])
                    bv = plsc.load_gather(bufb, [dr, dc])
                    al = av + bv
                    al = jnp.where(al >= 0.0, al, al * 0.2)
                    ex = jnp.exp(al)
                    abuf[pl.ds(j * 16, 16)] = ex
                    plsc.addupdate_scatter(dloc, [dr, dc], ex)
                    return carry2

                lax.fori_loop(0, CE // 16, p1body, 0)
                pltpu.sync_copy(abuf, ex_hbm[t * NH + h].at[pl.ds(base, CE)])
                return carry

            lax.fori_loop(0, EPT // CE, p1chunk, 0)
            pltpu.sync_copy(dloc, den_sp[h].at[idx80], add=True)

        plsc.subcore_barrier()

        for h in range(NH):
            pltpu.sync_copy(den_sp[h], dloc)
            zero_buf(bufa)

            def zcopy(j, carry):
                off = pl.multiple_of(row_start + j * ZR, ZR)
                pltpu.sync_copy(bufa.at[pl.ds(0, ZR)],
                                out_sp.at[pl.ds(off, ZR)])
                return carry

            lax.fori_loop(0, nchunks, zcopy, 0)
            plsc.subcore_barrier()

            def p2chunk(c, carry, h=h):
                base = pl.multiple_of(ebase + c * CE, 8)
                pltpu.sync_copy(src_h.at[pl.ds(base, CE)], srcb)
                pltpu.sync_copy(dst_h.at[pl.ds(base, CE)], dstb)
                pltpu.sync_copy(ex_hbm[t * NH + h].at[pl.ds(base, CE)], abuf)

                def nrm(j, carry2):
                    d16 = dstb[pl.ds(j * 16, 16)]
                    den = plsc.load_gather(
                        dloc, [lax.shift_right_logical(d16, 7),
                               jnp.bitwise_and(d16, 127)])
                    ex = abuf[pl.ds(j * 16, 16)]
                    abuf[pl.ds(j * 16, 16)] = ex / (den + 1e-16)
                    return carry2

                lax.fori_loop(0, CE // 16, nrm, 0)

                def sub(g, carry2, h=h):
                    @pl.when(g > 0)
                    def _():
                        pltpu.make_async_copy(bufb, out_sp.at[sidx],
                                              ssem).wait()

                    for u in range(KG // 16):
                        gidx[pl.ds(16 * u, 16)] = srcb[pl.ds(g * KG + 16 * u, 16)]
                        sidx[pl.ds(16 * u, 16)] = dstb[pl.ds(g * KG + 16 * u, 16)]
                    pltpu.async_copy(ht_h[h].at[gidx], bufa, sem).wait()

                    def scale(qg, carry3):
                        av16 = abuf[pl.ds(g * KG + qg * 16, 16)]
                        for i in range(16):
                            a = av16[i]
                            r = qg * 16 + i
                            for v in range(HW // 16):
                                bufb[r, pl.ds(16 * v, 16)] = (
                                    bufa[r, pl.ds(16 * v, 16)] * a
                                )
                        return carry3

                    lax.fori_loop(0, KG // 16, scale, 0)
                    pltpu.async_copy(bufb, out_sp.at[sidx], ssem, add=True)
                    return carry2

                lax.fori_loop(0, CE // KG, sub, 0)
                pltpu.make_async_copy(bufb, out_sp.at[sidx], ssem).wait()
                return carry

            lax.fori_loop(0, EPT // CE, p2chunk, 0)
            plsc.subcore_barrier()

            def dcopy(j, carry, h=h):
                off = pl.multiple_of(row_start + j * ZR, ZR)
                pltpu.sync_copy(out_sp.at[pl.ds(off, ZR)],
                                out_hbm.at[t, h, pl.ds(off, ZR)])
                return carry

            lax.fori_loop(0, nchunks, dcopy, 0)
            plsc.subcore_barrier()


    @pl.when(cid == 0)
    def _():
        run(0, src0, dst0, as0, ad0, ht0)

    @pl.when(cid == 1)
    def _():
        run(1, src1, dst1, as1, ad1, ht1)


def _sc_edge_conv(src0, dst0, as0, ad0, ht0, src1, dst1, as1, ad1, ht1):
    mesh = plsc.VectorSubcoreMesh(core_axis_name="c", subcore_axis_name="s",
                                  num_cores=2)
    f = pl.kernel(
        _sc_edge_conv_body,
        out_type=(
            jax.ShapeDtypeStruct((2, NHH, NA, HW), jnp.float32),
            [jax.ShapeDtypeStruct((NE,), jnp.float32) for _ in range(2 * NH)],
        ),
        mesh=mesh,
        compiler_params=pltpu.CompilerParams(needs_layout_passes=False),
        scratch_types=[
            pltpu.VMEM((DR, 128), jnp.float32),
            pltpu.VMEM((DR, 128), jnp.float32),
            pltpu.VMEM((DR, 128), jnp.float32),
            pltpu.VMEM((CE,), jnp.float32),
            pltpu.VMEM((CE,), jnp.int32),
            pltpu.VMEM((CE,), jnp.int32),
            pltpu.VMEM((KG,), jnp.int32),
            pltpu.VMEM((KG,), jnp.int32),
            pltpu.VMEM((DR,), jnp.int32),
            [pltpu.VMEM_SHARED((DR, 128), jnp.float32) for _ in range(NH)],
            pltpu.VMEM_SHARED((NA, HW), jnp.float32),
            pltpu.SemaphoreType.DMA,
            pltpu.SemaphoreType.DMA,
        ],
    )
    out, _ex = f(src0, dst0, as0, ad0, ht0, src1, dst1, as1, ad1, ht1)
    return out


BN2 = 1000


def _sem_body(o_ref, kw_ref, kb_ref, s_ref):
    i = pl.program_id(1)
    blk = jnp.concatenate([o_ref[0, j] for j in range(NHH)], axis=-1)
    blk = jnp.maximum(blk, 0.0)
    kk = jnp.tanh(
        jnp.dot(blk, kw_ref[...], preferred_element_type=jnp.float32)
        + kb_ref[...]
    )
    part = jnp.sum(kk, axis=0, keepdims=True)[None]

    @pl.when(i == 0)
    def _():
        s_ref[...] = jnp.zeros_like(s_ref)

    s_ref[...] += part


def _sem_sums(out_sc, k_w, k_b):
    return pl.pallas_call(
        _sem_body,
        grid=(2, NA // BN2),
        in_specs=[
            pl.BlockSpec((1, NHH, BN2, HW), lambda t, i: (t, 0, i, 0)),
            pl.BlockSpec((HID, HID), lambda t, i: (0, 0)),
            pl.BlockSpec((1, HID), lambda t, i: (0, 0)),
        ],
        out_specs=pl.BlockSpec((1, 1, HID), lambda t, i: (t, 0, 0)),
        out_shape=jax.ShapeDtypeStruct((2, 1, HID), jnp.float32),
    )(out_sc, k_w, k_b)


def _fuse_body(o_ref, s_ref, q_ref, lw_ref, lb_ref, logits_ref, hrep_ref):
    qv = q_ref[...]
    s0 = jnp.sum(qv[0] * s_ref[0, 0]) / NA
    s1 = jnp.sum(qv[0] * s_ref[1, 0]) / NA
    m = jnp.maximum(s0, s1)
    e0 = jnp.exp(s0 - m)
    e1 = jnp.exp(s1 - m)
    den = e0 + e1
    a0 = e0 / den
    a1 = e1 / den
    b0 = jnp.concatenate([o_ref[0, j] for j in range(NHH)], axis=-1)
    b1 = jnp.concatenate([o_ref[1, j] for j in range(NHH)], axis=-1)
    b0 = jnp.maximum(b0, 0.0)
    b1 = jnp.maximum(b1, 0.0)
    fused = a0 * b0 + a1 * b1
    logits_ref[...] = (
        jnp.dot(fused, lw_ref[...], preferred_element_type=jnp.float32)
        + lb_ref[...]
    )
    hrep_ref[...] = jnp.broadcast_to(fused[:, None, :], (BN2, 2, HID))


def _fuse(out_sc, s2, q, lin_w, lin_b):
    return pl.pallas_call(
        _fuse_body,
        grid=(NA // BN2,),
        in_specs=[
            pl.BlockSpec((2, NHH, BN2, HW), lambda i: (0, 0, i, 0)),
            pl.BlockSpec((2, 1, HID), lambda i: (0, 0, 0)),
            pl.BlockSpec((1, HID), lambda i: (0, 0)),
            pl.BlockSpec((HID, NCLS), lambda i: (0, 0)),
            pl.BlockSpec((1, NCLS), lambda i: (0, 0)),
        ],
        out_specs=[
            pl.BlockSpec((BN2, NCLS), lambda i: (i, 0)),
            pl.BlockSpec((BN2, 2, HID), lambda i: (i, 0, 0)),
        ],
        out_shape=[
            jax.ShapeDtypeStruct((NA, NCLS), jnp.float32),
            jax.ShapeDtypeStruct((NA, 2, HID), jnp.float32),
        ],
    )(out_sc, s2, q, lin_w, lin_b)


def _att_matrix(att_stack):
    att_t = jnp.transpose(att_stack, (1, 2, 0))
    w = att_t[:, :, :, None] * jnp.eye(NH, dtype=att_stack.dtype)[:, None, None, :]
    return w.reshape(HID, att_stack.shape[0] * NH)


def kernel(x_A, x_B, edge_index_ba, edge_index_aa, W_A, b_A, W_B, b_B,
           att_src_ba, att_dst_ba, att_src_aa, att_dst_aa, k_W, k_b, q,
           lin_W, lin_b):
    att_a = jnp.stack(
        [att_dst_ba[0], att_src_aa[0], att_dst_aa[0]]
    )
    att_b = att_src_ba
    watt_a = _att_matrix(att_a)
    watt_b = _att_matrix(att_b)

    xA_p = jnp.pad(x_A, ((0, NAP - NA), (0, 0)))
    xB_p = jnp.pad(x_B, ((0, NAP - NB), (0, 0)))
    outs_a = _project(xA_p, W_A, b_A.reshape(1, HID), watt_a)
    outs_b = _project(xB_p, W_B, b_B.reshape(1, HID), watt_b)
    htA = tuple(outs_a[:NHH])
    htB = tuple(outs_b[:NHH])
    avA = [a.reshape(DR, 128) for a in outs_a[NHH:]]
    avB = [a.reshape(DR, 128) for a in outs_b[NHH:]]
    ad_ba = tuple(avA[0:NH])
    as_aa = tuple(avA[NH:2 * NH])
    ad_aa = tuple(avA[2 * NH:3 * NH])
    as_ba = tuple(avB)

    out_sc = _sc_edge_conv(
        edge_index_ba[0], edge_index_ba[1], as_ba, ad_ba, htB,
        edge_index_aa[0], edge_index_aa[1], as_aa, ad_aa, htA,
    )

    s2 = _sem_sums(out_sc, k_W, k_b.reshape(1, HID))
    logits, hrep = _fuse(out_sc, s2, q.reshape(1, HID), lin_W,
                         lin_b.reshape(1, NCLS))

    alpha = jnp.full((NA, 2), 0.5, jnp.float32)
    return (logits, hrep, alpha)


# speedup vs baseline: 2.3618x; 1.4138x over previous
import functools

import jax
import jax.numpy as jnp
from jax import lax
from jax.experimental import pallas as pl
from jax.experimental.pallas import tpu as pltpu
from jax.experimental.pallas import tpu_sc as plsc

NA = 10000
NB = 10000
NE = 160000
IN_DIM = 256
HID = 512
NH = 4
HD = 128
NCLS = 40

NAP = 10240
NSUB = 16
EPT = NE // NSUB
RPT = 624
CE = 2000
KG = 80
ZR = 16
HW = 128
NSPLIT = HD // HW
NHH = NH * NSPLIT
DR = NAP // 128


BN1 = 512


def _proj_body(x_ref, w_ref, b_ref, watt_ref, *out_refs):
    ka = watt_ref.shape[1]
    ht_refs = out_refs[:NHH]
    av_refs = out_refs[NHH:]
    h = jnp.dot(x_ref[...], w_ref[...], preferred_element_type=jnp.float32)
    h = h + b_ref[...]
    for j in range(NHH):
        ht_refs[j][...] = h[:, j * HW:(j + 1) * HW]
    av = jnp.dot(h, watt_ref[...], preferred_element_type=jnp.float32)
    for j in range(ka):
        av_refs[j][...] = av[:, j:j + 1]


def _project(x, w, b, watt):
    n = x.shape[0]
    ka = watt.shape[1]
    grid = (n // BN1,)
    return pl.pallas_call(
        _proj_body,
        grid=grid,
        in_specs=[
            pl.BlockSpec((BN1, IN_DIM), lambda i: (i, 0)),
            pl.BlockSpec((IN_DIM, HID), lambda i: (0, 0)),
            pl.BlockSpec((1, HID), lambda i: (0, 0)),
            pl.BlockSpec((HID, ka), lambda i: (0, 0)),
        ],
        out_specs=(
            [pl.BlockSpec((BN1, HW), lambda i: (i, 0)) for _ in range(NHH)]
            + [pl.BlockSpec((BN1, 1), lambda i: (i, 0)) for _ in range(ka)]
        ),
        out_shape=(
            [jax.ShapeDtypeStruct((n, HW), jnp.float32) for _ in range(NHH)]
            + [jax.ShapeDtypeStruct((n, 1), jnp.float32) for _ in range(ka)]
        ),
    )(x, w, b, watt)


def _sc_edge_conv_body(
    src0, dst0, as0, ad0, ht0,
    src1, dst1, as1, ad1, ht1,
    out_hbm, ex_hbm,
    bufa, bufb, dloc, abuf, srcb, dstb, gidx, gidxb, sidx, idx80,
    den_sp, out_sp, sem, ssem,
):
    cid = lax.axis_index("c")
    sid = lax.axis_index("s")
    zero16 = jnp.zeros((16,), jnp.float32)
    lanes = lax.iota(jnp.int32, 16)

    def zero_buf(buf):
        def zrow(r, carry):
            for v in range(HW // 16):
                buf[r, pl.ds(16 * v, 16)] = zero16
            return carry

        lax.fori_loop(0, KG, zrow, 0)

    def zero_dloc():
        def zrow(r, carry):
            for v in range(8):
                dloc[r, pl.ds(16 * v, 16)] = zero16
            return carry

        lax.fori_loop(0, DR, zrow, 0)

    def run(t, src_h, dst_h, as_h, ad_h, ht_h):
        ebase = sid * EPT
        row_start = sid * RPT
        nchunks = jnp.where(sid == NSUB - 1, (NA - RPT * (NSUB - 1)) // ZR,
                            RPT // ZR)

        zero_dloc()

        @pl.when(sid == 0)
        def _():
            for h in range(NH):
                pltpu.sync_copy(dloc, den_sp[h])

        for u in range(DR // 16):
            idx80[pl.ds(16 * u, 16)] = lanes + (16 * u)
        plsc.subcore_barrier()

        for h in range(NH):
            pltpu.sync_copy(as_h[h], bufa)
            pltpu.sync_copy(ad_h[h], bufb)
            zero_dloc()

            def p1chunk(c, carry, h=h):
                base = pl.multiple_of(ebase + c * CE, 8)
                pltpu.sync_copy(src_h.at[pl.ds(base, CE)], srcb)
                pltpu.sync_copy(dst_h.at[pl.ds(base, CE)], dstb)

                def p1body(j, carry2):
                    s16 = srcb[pl.ds(j * 16, 16)]
                    d16 = dstb[pl.ds(j * 16, 16)]
                    sr = lax.shift_right_logical(s16, 7)
                    sc = jnp.bitwise_and(s16, 127)
                    dr = lax.shift_right_logical(d16, 7)
                    dc = jnp.bitwise_and(d16, 127)
                    av = plsc.load_gather(bufa, [sr, sc])
                    bv = plsc.load_gather(bufb, [dr, dc])
                    al = av + bv
                    al = jnp.where(al >= 0.0, al, al * 0.2)
                    ex = jnp.exp(al)
                    abuf[pl.ds(j * 16, 16)] = ex
                    plsc.addupdate_scatter(dloc, [dr, dc], ex)
                    return carry2

                lax.fori_loop(0, CE // 16, p1body, 0)
                pltpu.sync_copy(abuf, ex_hbm[t * NH + h].at[pl.ds(base, CE)])
                return carry

            lax.fori_loop(0, EPT // CE, p1chunk, 0)
            pltpu.sync_copy(dloc, den_sp[h].at[idx80], add=True)

        plsc.subcore_barrier()

        for h in range(NH):
            pltpu.sync_copy(den_sp[h], dloc)
            zero_buf(bufa)

            def zcopy(j, carry):
                off = pl.multiple_of(row_start + j * ZR, ZR)
                pltpu.sync_copy(bufa.at[pl.ds(0, ZR)],
                                out_sp.at[pl.ds(off, ZR)])
                return carry

            lax.fori_loop(0, nchunks, zcopy, 0)
            plsc.subcore_barrier()

            def p2chunk(c, carry, h=h):
                base = pl.multiple_of(ebase + c * CE, 8)
                pltpu.sync_copy(src_h.at[pl.ds(base, CE)], srcb)
                pltpu.sync_copy(dst_h.at[pl.ds(base, CE)], dstb)
                pltpu.sync_copy(ex_hbm[t * NH + h].at[pl.ds(base, CE)], abuf)

                nsub = CE // KG

                def fire(g, buf, gi, sm, h=h):
                    for u in range(KG // 16):
                        gi[pl.ds(16 * u, 16)] = srcb[pl.ds(g * KG + 16 * u, 16)]
                    pltpu.async_copy(ht_h[h].at[gi], buf, sm)

                fire(0, bufa, gidx, sem)

                def nrm(j, carry2):
                    d16 = dstb[pl.ds(j * 16, 16)]
                    den = plsc.load_gather(
                        dloc, [lax.shift_right_logical(d16, 7),
                               jnp.bitwise_and(d16, 127)])
                    ex = abuf[pl.ds(j * 16, 16)]
                    abuf[pl.ds(j * 16, 16)] = ex / (den + 1e-16)
                    return carry2

                lax.fori_loop(0, CE // 16, nrm, 0)

                def consume(g, buf, gi, sm):
                    pltpu.make_async_copy(ht_h[h].at[gi], buf, sm).wait()

                    def scale(r, carry3):
                        a = plsc.load_gather(
                            abuf, [lanes * 0 + (g * KG + r)])
                        for v in range(HW // 16):
                            buf[r, pl.ds(16 * v, 16)] = (
                                buf[r, pl.ds(16 * v, 16)] * a
                            )
                        return carry3

                    lax.fori_loop(0, KG, scale, 0)
                    for u in range(KG // 16):
                        sidx[pl.ds(16 * u, 16)] = dstb[pl.ds(g * KG + 16 * u, 16)]
                    pltpu.sync_copy(buf, out_sp.at[sidx], add=True)

                def sub2(gg, carry2, h=h):
                    g0 = gg * 2
                    fire(g0 + 1, bufb, gidxb, ssem)
                    consume(g0, bufa, gidx, sem)
                    fire(g0 + 2, bufa, gidx, sem)
                    consume(g0 + 1, bufb, gidxb, ssem)
                    return carry2

                lax.fori_loop(0, (nsub - 1) // 2, sub2, 0)
                consume(nsub - 1, bufa, gidx, sem)
                return carry

            lax.fori_loop(0, EPT // CE, p2chunk, 0)
            plsc.subcore_barrier()

            def dcopy(j, carry, h=h):
                off = pl.multiple_of(row_start + j * ZR, ZR)
                pltpu.sync_copy(out_sp.at[pl.ds(off, ZR)],
                                out_hbm.at[t, h, pl.ds(off, ZR)])
                return carry

            lax.fori_loop(0, nchunks, dcopy, 0)
            plsc.subcore_barrier()


    @pl.when(cid == 0)
    def _():
        run(0, src0, dst0, as0, ad0, ht0)

    @pl.when(cid == 1)
    def _():
        run(1, src1, dst1, as1, ad1, ht1)


def _sc_edge_conv(src0, dst0, as0, ad0, ht0, src1, dst1, as1, ad1, ht1):
    mesh = plsc.VectorSubcoreMesh(core_axis_name="c", subcore_axis_name="s",
                                  num_cores=2)
    f = pl.kernel(
        _sc_edge_conv_body,
        out_type=(
            jax.ShapeDtypeStruct((2, NHH, NA, HW), jnp.float32),
            [jax.ShapeDtypeStruct((NE,), jnp.float32) for _ in range(2 * NH)],
        ),
        mesh=mesh,
        compiler_params=pltpu.CompilerParams(needs_layout_passes=False),
        scratch_types=[
            pltpu.VMEM((DR, 128), jnp.float32),
            pltpu.VMEM((DR, 128), jnp.float32),
            pltpu.VMEM((DR, 128), jnp.float32),
            pltpu.VMEM((CE,), jnp.float32),
            pltpu.VMEM((CE,), jnp.int32),
            pltpu.VMEM((CE,), jnp.int32),
            pltpu.VMEM((KG,), jnp.int32),
            pltpu.VMEM((KG,), jnp.int32),
            pltpu.VMEM((KG,), jnp.int32),
            pltpu.VMEM((DR,), jnp.int32),
            [pltpu.VMEM_SHARED((DR, 128), jnp.float32) for _ in range(NH)],
            pltpu.VMEM_SHARED((NA, HW), jnp.float32),
            pltpu.SemaphoreType.DMA,
            pltpu.SemaphoreType.DMA,
        ],
    )
    out, _ex = f(src0, dst0, as0, ad0, ht0, src1, dst1, as1, ad1, ht1)
    return out


BN2 = 1000


def _sem_body(o_ref, kw_ref, kb_ref, s_ref):
    i = pl.program_id(1)
    blk = jnp.concatenate([o_ref[0, j] for j in range(NHH)], axis=-1)
    blk = jnp.maximum(blk, 0.0)
    kk = jnp.tanh(
        jnp.dot(blk, kw_ref[...], preferred_element_type=jnp.float32)
        + kb_ref[...]
    )
    part = jnp.sum(kk, axis=0, keepdims=True)[None]

    @pl.when(i == 0)
    def _():
        s_ref[...] = jnp.zeros_like(s_ref)

    s_ref[...] += part


def _sem_sums(out_sc, k_w, k_b):
    return pl.pallas_call(
        _sem_body,
        grid=(2, NA // BN2),
        in_specs=[
            pl.BlockSpec((1, NHH, BN2, HW), lambda t, i: (t, 0, i, 0)),
            pl.BlockSpec((HID, HID), lambda t, i: (0, 0)),
            pl.BlockSpec((1, HID), lambda t, i: (0, 0)),
        ],
        out_specs=pl.BlockSpec((1, 1, HID), lambda t, i: (t, 0, 0)),
        out_shape=jax.ShapeDtypeStruct((2, 1, HID), jnp.float32),
    )(out_sc, k_w, k_b)


def _fuse_body(o_ref, s_ref, q_ref, lw_ref, lb_ref, logits_ref, hrep_ref):
    qv = q_ref[...]
    s0 = jnp.sum(qv[0] * s_ref[0, 0]) / NA
    s1 = jnp.sum(qv[0] * s_ref[1, 0]) / NA
    m = jnp.maximum(s0, s1)
    e0 = jnp.exp(s0 - m)
    e1 = jnp.exp(s1 - m)
    den = e0 + e1
    a0 = e0 / den
    a1 = e1 / den
    b0 = jnp.concatenate([o_ref[0, j] for j in range(NHH)], axis=-1)
    b1 = jnp.concatenate([o_ref[1, j] for j in range(NHH)], axis=-1)
    b0 = jnp.maximum(b0, 0.0)
    b1 = jnp.maximum(b1, 0.0)
    fused = a0 * b0 + a1 * b1
    logits_ref[...] = (
        jnp.dot(fused, lw_ref[...], preferred_element_type=jnp.float32)
        + lb_ref[...]
    )
    hrep_ref[...] = jnp.broadcast_to(fused[:, None, :], (BN2, 2, HID))


def _fuse(out_sc, s2, q, lin_w, lin_b):
    return pl.pallas_call(
        _fuse_body,
        grid=(NA // BN2,),
        in_specs=[
            pl.BlockSpec((2, NHH, BN2, HW), lambda i: (0, 0, i, 0)),
            pl.BlockSpec((2, 1, HID), lambda i: (0, 0, 0)),
            pl.BlockSpec((1, HID), lambda i: (0, 0)),
            pl.BlockSpec((HID, NCLS), lambda i: (0, 0)),
            pl.BlockSpec((1, NCLS), lambda i: (0, 0)),
        ],
        out_specs=[
            pl.BlockSpec((BN2, NCLS), lambda i: (i, 0)),
            pl.BlockSpec((BN2, 2, HID), lambda i: (i, 0, 0)),
        ],
        out_shape=[
            jax.ShapeDtypeStruct((NA, NCLS), jnp.float32),
            jax.ShapeDtypeStruct((NA, 2, HID), jnp.float32),
        ],
    )(out_sc, s2, q, lin_w, lin_b)


def _att_matrix(att_stack):
    att_t = jnp.transpose(att_stack, (1, 2, 0))
    w = att_t[:, :, :, None] * jnp.eye(NH, dtype=att_stack.dtype)[:, None, None, :]
    return w.reshape(HID, att_stack.shape[0] * NH)


def kernel(x_A, x_B, edge_index_ba, edge_index_aa, W_A, b_A, W_B, b_B,
           att_src_ba, att_dst_ba, att_src_aa, att_dst_aa, k_W, k_b, q,
           lin_W, lin_b):
    att_a = jnp.stack(
        [att_dst_ba[0], att_src_aa[0], att_dst_aa[0]]
    )
    att_b = att_src_ba
    watt_a = _att_matrix(att_a)
    watt_b = _att_matrix(att_b)

    xA_p = jnp.pad(x_A, ((0, NAP - NA), (0, 0)))
    xB_p = jnp.pad(x_B, ((0, NAP - NB), (0, 0)))
    outs_a = _project(xA_p, W_A, b_A.reshape(1, HID), watt_a)
    outs_b = _project(xB_p, W_B, b_B.reshape(1, HID), watt_b)
    htA = tuple(outs_a[:NHH])
    htB = tuple(outs_b[:NHH])
    avA = [a.reshape(DR, 128) for a in outs_a[NHH:]]
    avB = [a.reshape(DR, 128) for a in outs_b[NHH:]]
    ad_ba = tuple(avA[0:NH])
    as_aa = tuple(avA[NH:2 * NH])
    ad_aa = tuple(avA[2 * NH:3 * NH])
    as_ba = tuple(avB)

    out_sc = _sc_edge_conv(
        edge_index_ba[0], edge_index_ba[1], as_ba, ad_ba, htB,
        edge_index_aa[0], edge_index_aa[1], as_aa, ad_aa, htA,
    )

    s2 = _sem_sums(out_sc, k_W, k_b.reshape(1, HID))
    logits, hrep = _fuse(out_sc, s2, q.reshape(1, HID), lin_W,
                         lin_b.reshape(1, NCLS))

    alpha = jnp.full((NA, 2), 0.5, jnp.float32)
    return (logits, hrep, alpha)


# speedup vs baseline: 2.6638x; 1.1279x over previous
import functools

import jax
import jax.numpy as jnp
from jax import lax
from jax.experimental import pallas as pl
from jax.experimental.pallas import tpu as pltpu
from jax.experimental.pallas import tpu_sc as plsc

NA = 10000
NB = 10000
NE = 160000
IN_DIM = 256
HID = 512
NH = 4
HD = 128
NCLS = 40

NAP = 10240
NSUB = 16
EPT = NE // NSUB
RPT = 624
CE = 2000
KG = 80
ZR = 16
HW = 128
NSPLIT = HD // HW
NHH = NH * NSPLIT
DR = NAP // 128


BN1 = 512


def _proj_body(x_ref, w_ref, b_ref, watt_ref, *out_refs):
    ka = watt_ref.shape[1]
    ht_refs = out_refs[:NHH]
    av_refs = out_refs[NHH:]
    h = jnp.dot(x_ref[...], w_ref[...], preferred_element_type=jnp.float32)
    h = h + b_ref[...]
    for j in range(NHH):
        ht_refs[j][...] = h[:, j * HW:(j + 1) * HW]
    av = jnp.dot(h, watt_ref[...], preferred_element_type=jnp.float32)
    for j in range(ka):
        av_refs[j][...] = av[:, j:j + 1]


def _project(x, w, b, watt):
    n = x.shape[0]
    ka = watt.shape[1]
    grid = (n // BN1,)
    return pl.pallas_call(
        _proj_body,
        grid=grid,
        in_specs=[
            pl.BlockSpec((BN1, IN_DIM), lambda i: (i, 0)),
            pl.BlockSpec((IN_DIM, HID), lambda i: (0, 0)),
            pl.BlockSpec((1, HID), lambda i: (0, 0)),
            pl.BlockSpec((HID, ka), lambda i: (0, 0)),
        ],
        out_specs=(
            [pl.BlockSpec((BN1, HW), lambda i: (i, 0)) for _ in range(NHH)]
            + [pl.BlockSpec((BN1, 1), lambda i: (i, 0)) for _ in range(ka)]
        ),
        out_shape=(
            [jax.ShapeDtypeStruct((n, HW), jnp.float32) for _ in range(NHH)]
            + [jax.ShapeDtypeStruct((n, 1), jnp.float32) for _ in range(ka)]
        ),
    )(x, w, b, watt)


def _sc_edge_conv_body(
    src0, dst0, as0, ad0, ht0,
    src1, dst1, as1, ad1, ht1,
    out_hbm, ex_hbm,
    bufa, bufb, dloc, abuf, srcb, dstb, gidx, gidxb, sidx, idx80,
    den_sp, out_sp, sem, ssem,
):
    cid = lax.axis_index("c")
    sid = lax.axis_index("s")
    zero16 = jnp.zeros((16,), jnp.float32)
    lanes = lax.iota(jnp.int32, 16)

    def zero_buf(buf):
        def zrow(r, carry):
            for v in range(HW // 16):
                buf[r, pl.ds(16 * v, 16)] = zero16
            return carry

        lax.fori_loop(0, KG, zrow, 0)

    def zero_dloc():
        def zrow(r, carry):
            for v in range(8):
                dloc[r, pl.ds(16 * v, 16)] = zero16
            return carry

        lax.fori_loop(0, DR, zrow, 0)

    def run(t, src_h, dst_h, as_h, ad_h, ht_h):
        ebase = sid * EPT
        row_start = sid * RPT
        nchunks = jnp.where(sid == NSUB - 1, (NA - RPT * (NSUB - 1)) // ZR,
                            RPT // ZR)

        zero_dloc()

        @pl.when(sid == 0)
        def _():
            for h in range(NH):
                pltpu.sync_copy(dloc, den_sp[h])

        for u in range(DR // 16):
            idx80[pl.ds(16 * u, 16)] = lanes + (16 * u)
        plsc.subcore_barrier()

        for h in range(NH):
            pltpu.sync_copy(as_h[h], bufa)
            pltpu.sync_copy(ad_h[h], bufb)
            zero_dloc()

            def p1chunk(c, carry, h=h):
                base = pl.multiple_of(ebase + c * CE, 8)
                pltpu.sync_copy(src_h.at[pl.ds(base, CE)], srcb)
                pltpu.sync_copy(dst_h.at[pl.ds(base, CE)], dstb)

                def p1body(j, carry2):
                    s16 = srcb[pl.ds(j * 16, 16)]
                    d16 = dstb[pl.ds(j * 16, 16)]
                    sr = lax.shift_right_logical(s16, 7)
                    sc = jnp.bitwise_and(s16, 127)
                    dr = lax.shift_right_logical(d16, 7)
                    dc = jnp.bitwise_and(d16, 127)
                    av = plsc.load_gather(bufa, [sr, sc])
                    bv = plsc.load_gather(bufb, [dr, dc])
                    al = av + bv
                    al = jnp.where(al >= 0.0, al, al * 0.2)
                    ex = jnp.exp(al)
                    abuf[pl.ds(j * 16, 16)] = ex
                    plsc.addupdate_scatter(dloc, [dr, dc], ex)
                    return carry2

                lax.fori_loop(0, CE // 16, p1body, 0)
                pltpu.sync_copy(abuf, ex_hbm[t * NH + h].at[pl.ds(base, CE)])
                return carry

            lax.fori_loop(0, EPT // CE, p1chunk, 0)
            pltpu.sync_copy(dloc, den_sp[h].at[idx80], add=True)

        plsc.subcore_barrier()

        for h in range(NH):
            pltpu.sync_copy(den_sp[h], dloc)
            zero_buf(bufa)

            def zcopy(j, carry):
                off = pl.multiple_of(row_start + j * ZR, ZR)
                pltpu.sync_copy(bufa.at[pl.ds(0, ZR)],
                                out_sp.at[pl.ds(off, ZR)])
                return carry

            lax.fori_loop(0, nchunks, zcopy, 0)
            plsc.subcore_barrier()

            def p2chunk(c, carry, h=h):
                base = pl.multiple_of(ebase + c * CE, 8)
                pltpu.sync_copy(src_h.at[pl.ds(base, CE)], srcb)
                pltpu.sync_copy(dst_h.at[pl.ds(base, CE)], dstb)
                pltpu.sync_copy(ex_hbm[t * NH + h].at[pl.ds(base, CE)], abuf)

                nsub = CE // KG

                def fire(g, buf, gi, sm, h=h):
                    for u in range(KG // 16):
                        gi[pl.ds(16 * u, 16)] = srcb[pl.ds(g * KG + 16 * u, 16)]
                    pltpu.async_copy(ht_h[h].at[gi], buf, sm)

                fire(0, bufa, gidx, sem)

                def nrm(j, carry2):
                    d16 = dstb[pl.ds(j * 16, 16)]
                    den = plsc.load_gather(
                        dloc, [lax.shift_right_logical(d16, 7),
                               jnp.bitwise_and(d16, 127)])
                    ex = abuf[pl.ds(j * 16, 16)]
                    abuf[pl.ds(j * 16, 16)] = ex / (den + 1e-16)
                    return carry2

                lax.fori_loop(0, CE // 16, nrm, 0)

                def consume(g, buf, gi, sm, fast):
                    pltpu.make_async_copy(ht_h[h].at[gi], buf, sm).wait()

                    if fast:
                        def scale(qg, carry3):
                            av16 = abuf[pl.ds(g * KG + qg * 16, 16)]
                            for i in range(16):
                                a = av16[i]
                                r = qg * 16 + i
                                for v in range(HW // 16):
                                    buf[r, pl.ds(16 * v, 16)] = (
                                        buf[r, pl.ds(16 * v, 16)] * a
                                    )
                            return carry3

                        lax.fori_loop(0, KG // 16, scale, 0)
                    else:
                        def scale(r, carry3):
                            a = plsc.load_gather(
                                abuf, [lanes * 0 + (g * KG + r)])
                            for v in range(HW // 16):
                                buf[r, pl.ds(16 * v, 16)] = (
                                    buf[r, pl.ds(16 * v, 16)] * a
                                )
                            return carry3

                        lax.fori_loop(0, KG, scale, 0)
                    for u in range(KG // 16):
                        sidx[pl.ds(16 * u, 16)] = dstb[pl.ds(g * KG + 16 * u, 16)]
                    pltpu.sync_copy(buf, out_sp.at[sidx], add=True)

                def sub2(gg, carry2, h=h):
                    g0 = gg * 2
                    fire(g0 + 1, bufb, gidxb, ssem)
                    consume(g0, bufa, gidx, sem, True)
                    fire(g0 + 2, bufa, gidx, sem)
                    consume(g0 + 1, bufb, gidxb, ssem, True)
                    return carry2

                lax.fori_loop(0, (nsub - 1) // 2, sub2, 0)
                consume(nsub - 1, bufa, gidx, sem, False)
                return carry

            lax.fori_loop(0, EPT // CE, p2chunk, 0)
            plsc.subcore_barrier()

            def dcopy(j, carry, h=h):
                off = pl.multiple_of(row_start + j * ZR, ZR)
                pltpu.sync_copy(out_sp.at[pl.ds(off, ZR)],
                                out_hbm.at[t, h, pl.ds(off, ZR)])
                return carry

            lax.fori_loop(0, nchunks, dcopy, 0)
            plsc.subcore_barrier()


    @pl.when(cid == 0)
    def _():
        run(0, src0, dst0, as0, ad0, ht0)

    @pl.when(cid == 1)
    def _():
        run(1, src1, dst1, as1, ad1, ht1)


def _sc_edge_conv(src0, dst0, as0, ad0, ht0, src1, dst1, as1, ad1, ht1):
    mesh = plsc.VectorSubcoreMesh(core_axis_name="c", subcore_axis_name="s",
                                  num_cores=2)
    f = pl.kernel(
        _sc_edge_conv_body,
        out_type=(
            jax.ShapeDtypeStruct((2, NHH, NA, HW), jnp.float32),
            [jax.ShapeDtypeStruct((NE,), jnp.float32) for _ in range(2 * NH)],
        ),
        mesh=mesh,
        compiler_params=pltpu.CompilerParams(needs_layout_passes=False),
        scratch_types=[
            pltpu.VMEM((DR, 128), jnp.float32),
            pltpu.VMEM((DR, 128), jnp.float32),
            pltpu.VMEM((DR, 128), jnp.float32),
            pltpu.VMEM((CE,), jnp.float32),
            pltpu.VMEM((CE,), jnp.int32),
            pltpu.VMEM((CE,), jnp.int32),
            pltpu.VMEM((KG,), jnp.int32),
            pltpu.VMEM((KG,), jnp.int32),
            pltpu.VMEM((KG,), jnp.int32),
            pltpu.VMEM((DR,), jnp.int32),
            [pltpu.VMEM_SHARED((DR, 128), jnp.float32) for _ in range(NH)],
            pltpu.VMEM_SHARED((NA, HW), jnp.float32),
            pltpu.SemaphoreType.DMA,
            pltpu.SemaphoreType.DMA,
        ],
    )
    out, _ex = f(src0, dst0, as0, ad0, ht0, src1, dst1, as1, ad1, ht1)
    return out


BN2 = 1000


def _sem_body(o_ref, kw_ref, kb_ref, s_ref):
    i = pl.program_id(1)
    blk = jnp.concatenate([o_ref[0, j] for j in range(NHH)], axis=-1)
    blk = jnp.maximum(blk, 0.0)
    kk = jnp.tanh(
        jnp.dot(blk, kw_ref[...], preferred_element_type=jnp.float32)
        + kb_ref[...]
    )
    part = jnp.sum(kk, axis=0, keepdims=True)[None]

    @pl.when(i == 0)
    def _():
        s_ref[...] = jnp.zeros_like(s_ref)

    s_ref[...] += part


def _sem_sums(out_sc, k_w, k_b):
    return pl.pallas_call(
        _sem_body,
        grid=(2, NA // BN2),
        in_specs=[
            pl.BlockSpec((1, NHH, BN2, HW), lambda t, i: (t, 0, i, 0)),
            pl.BlockSpec((HID, HID), lambda t, i: (0, 0)),
            pl.BlockSpec((1, HID), lambda t, i: (0, 0)),
        ],
        out_specs=pl.BlockSpec((1, 1, HID), lambda t, i: (t, 0, 0)),
        out_shape=jax.ShapeDtypeStruct((2, 1, HID), jnp.float32),
    )(out_sc, k_w, k_b)


def _fuse_body(o_ref, s_ref, q_ref, lw_ref, lb_ref, logits_ref, hrep_ref):
    qv = q_ref[...]
    s0 = jnp.sum(qv[0] * s_ref[0, 0]) / NA
    s1 = jnp.sum(qv[0] * s_ref[1, 0]) / NA
    m = jnp.maximum(s0, s1)
    e0 = jnp.exp(s0 - m)
    e1 = jnp.exp(s1 - m)
    den = e0 + e1
    a0 = e0 / den
    a1 = e1 / den
    b0 = jnp.concatenate([o_ref[0, j] for j in range(NHH)], axis=-1)
    b1 = jnp.concatenate([o_ref[1, j] for j in range(NHH)], axis=-1)
    b0 = jnp.maximum(b0, 0.0)
    b1 = jnp.maximum(b1, 0.0)
    fused = a0 * b0 + a1 * b1
    logits_ref[...] = (
        jnp.dot(fused, lw_ref[...], preferred_element_type=jnp.float32)
        + lb_ref[...]
    )
    hrep_ref[...] = jnp.broadcast_to(fused[:, None, :], (BN2, 2, HID))


def _fuse(out_sc, s2, q, lin_w, lin_b):
    return pl.pallas_call(
        _fuse_body,
        grid=(NA // BN2,),
        in_specs=[
            pl.BlockSpec((2, NHH, BN2, HW), lambda i: (0, 0, i, 0)),
            pl.BlockSpec((2, 1, HID), lambda i: (0, 0, 0)),
            pl.BlockSpec((1, HID), lambda i: (0, 0)),
            pl.BlockSpec((HID, NCLS), lambda i: (0, 0)),
            pl.BlockSpec((1, NCLS), lambda i: (0, 0)),
        ],
        out_specs=[
            pl.BlockSpec((BN2, NCLS), lambda i: (i, 0)),
            pl.BlockSpec((BN2, 2, HID), lambda i: (i, 0, 0)),
        ],
        out_shape=[
            jax.ShapeDtypeStruct((NA, NCLS), jnp.float32),
            jax.ShapeDtypeStruct((NA, 2, HID), jnp.float32),
        ],
    )(out_sc, s2, q, lin_w, lin_b)


def _att_matrix(att_stack):
    att_t = jnp.transpose(att_stack, (1, 2, 0))
    w = att_t[:, :, :, None] * jnp.eye(NH, dtype=att_stack.dtype)[:, None, None, :]
    return w.reshape(HID, att_stack.shape[0] * NH)


def kernel(x_A, x_B, edge_index_ba, edge_index_aa, W_A, b_A, W_B, b_B,
           att_src_ba, att_dst_ba, att_src_aa, att_dst_aa, k_W, k_b, q,
           lin_W, lin_b):
    att_a = jnp.stack(
        [att_dst_ba[0], att_src_aa[0], att_dst_aa[0]]
    )
    att_b = att_src_ba
    watt_a = _att_matrix(att_a)
    watt_b = _att_matrix(att_b)

    xA_p = jnp.pad(x_A, ((0, NAP - NA), (0, 0)))
    xB_p = jnp.pad(x_B, ((0, NAP - NB), (0, 0)))
    outs_a = _project(xA_p, W_A, b_A.reshape(1, HID), watt_a)
    outs_b = _project(xB_p, W_B, b_B.reshape(1, HID), watt_b)
    htA = tuple(outs_a[:NHH])
    htB = tuple(outs_b[:NHH])
    avA = [a.reshape(DR, 128) for a in outs_a[NHH:]]
    avB = [a.reshape(DR, 128) for a in outs_b[NHH:]]
    ad_ba = tuple(avA[0:NH])
    as_aa = tuple(avA[NH:2 * NH])
    ad_aa = tuple(avA[2 * NH:3 * NH])
    as_ba = tuple(avB)

    out_sc = _sc_edge_conv(
        edge_index_ba[0], edge_index_ba[1], as_ba, ad_ba, htB,
        edge_index_aa[0], edge_index_aa[1], as_aa, ad_aa, htA,
    )

    s2 = _sem_sums(out_sc, k_W, k_b.reshape(1, HID))
    logits, hrep = _fuse(out_sc, s2, q.reshape(1, HID), lin_W,
                         lin_b.reshape(1, NCLS))

    alpha = jnp.full((NA, 2), 0.5, jnp.float32)
    return (logits, hrep, alpha)


# speedup vs baseline: 2.6860x; 1.0083x over previous
import functools

import jax
import jax.numpy as jnp
from jax import lax
from jax.experimental import pallas as pl
from jax.experimental.pallas import tpu as pltpu
from jax.experimental.pallas import tpu_sc as plsc

NA = 10000
NB = 10000
NE = 160000
IN_DIM = 256
HID = 512
NH = 4
HD = 128
NCLS = 40

NAP = 10240
NSUB = 16
EPT = NE // NSUB
RPT = 624
CE = 2000
KG = 80
ZR = 16
HW = 128
NSPLIT = HD // HW
NHH = NH * NSPLIT
DR = NAP // 128


BN1 = 512


def _proj_body(x_ref, w_ref, b_ref, watt_ref, *out_refs):
    ka = watt_ref.shape[1]
    ht_refs = out_refs[:NHH]
    av_refs = out_refs[NHH:]
    h = jnp.dot(x_ref[...], w_ref[...], preferred_element_type=jnp.float32)
    h = h + b_ref[...]
    for j in range(NHH):
        ht_refs[j][...] = h[:, j * HW:(j + 1) * HW]
    av = jnp.dot(h, watt_ref[...], preferred_element_type=jnp.float32)
    for j in range(ka):
        av_refs[j][...] = av[:, j:j + 1]


def _project(x, w, b, watt):
    n = x.shape[0]
    ka = watt.shape[1]
    grid = (n // BN1,)
    return pl.pallas_call(
        _proj_body,
        grid=grid,
        in_specs=[
            pl.BlockSpec((BN1, IN_DIM), lambda i: (i, 0)),
            pl.BlockSpec((IN_DIM, HID), lambda i: (0, 0)),
            pl.BlockSpec((1, HID), lambda i: (0, 0)),
            pl.BlockSpec((HID, ka), lambda i: (0, 0)),
        ],
        out_specs=(
            [pl.BlockSpec((BN1, HW), lambda i: (i, 0)) for _ in range(NHH)]
            + [pl.BlockSpec((BN1, 1), lambda i: (i, 0)) for _ in range(ka)]
        ),
        out_shape=(
            [jax.ShapeDtypeStruct((n, HW), jnp.float32) for _ in range(NHH)]
            + [jax.ShapeDtypeStruct((n, 1), jnp.float32) for _ in range(ka)]
        ),
    )(x, w, b, watt)


def _sc_edge_conv_body(
    src0, dst0, as0, ad0, ht0,
    src1, dst1, as1, ad1, ht1,
    out_hbm, ex_hbm,
    bufa, bufb, dloc, abuf, srcb, dstb, gidx, gidxb, sidx, idx80,
    den_sp, out_sp, sem, ssem,
):
    cid = lax.axis_index("c")
    sid = lax.axis_index("s")
    zero16 = jnp.zeros((16,), jnp.float32)
    lanes = lax.iota(jnp.int32, 16)

    def zero_buf(buf):
        def zrow(r, carry):
            for v in range(HW // 16):
                buf[r, pl.ds(16 * v, 16)] = zero16
            return carry

        lax.fori_loop(0, KG, zrow, 0)

    def zero_dloc():
        def zrow(r, carry):
            for v in range(8):
                dloc[r, pl.ds(16 * v, 16)] = zero16
            return carry

        lax.fori_loop(0, DR, zrow, 0)

    def run(t, src_h, dst_h, as_h, ad_h, ht_h):
        ebase = sid * EPT
        row_start = sid * RPT
        nchunks = jnp.where(sid == NSUB - 1, (NA - RPT * (NSUB - 1)) // ZR,
                            RPT // ZR)

        zero_dloc()

        @pl.when(sid == 0)
        def _():
            for h in range(NH):
                pltpu.sync_copy(dloc, den_sp[h])

        for u in range(DR // 16):
            idx80[pl.ds(16 * u, 16)] = lanes + (16 * u)
        plsc.subcore_barrier()

        for h in range(NH):
            pltpu.sync_copy(as_h[h], bufa)
            pltpu.sync_copy(ad_h[h], bufb)
            zero_dloc()

            def p1chunk(c, carry, h=h):
                base = pl.multiple_of(ebase + c * CE, 8)
                pltpu.sync_copy(src_h.at[pl.ds(base, CE)], srcb)
                pltpu.sync_copy(dst_h.at[pl.ds(base, CE)], dstb)

                def p1body(j, carry2):
                    s16 = srcb[pl.ds(j * 16, 16)]
                    d16 = dstb[pl.ds(j * 16, 16)]
                    sr = lax.shift_right_logical(s16, 7)
                    sc = jnp.bitwise_and(s16, 127)
                    dr = lax.shift_right_logical(d16, 7)
                    dc = jnp.bitwise_and(d16, 127)
                    av = plsc.load_gather(bufa, [sr, sc])
                    bv = plsc.load_gather(bufb, [dr, dc])
                    al = av + bv
                    al = jnp.where(al >= 0.0, al, al * 0.2)
                    ex = jnp.exp(al)
                    abuf[pl.ds(j * 16, 16)] = ex
                    plsc.addupdate_scatter(dloc, [dr, dc], ex)
                    return carry2

                lax.fori_loop(0, CE // 16, p1body, 0)
                pltpu.sync_copy(abuf, ex_hbm[t * NH + h].at[pl.ds(base, CE)])
                return carry

            lax.fori_loop(0, EPT // CE, p1chunk, 0)
            pltpu.sync_copy(dloc, den_sp[h].at[idx80], add=True)

        plsc.subcore_barrier()

        for h in range(NH):
            pltpu.sync_copy(den_sp[h], dloc)
            zero_buf(bufa)

            def zcopy(j, carry):
                off = pl.multiple_of(row_start + j * ZR, ZR)
                pltpu.sync_copy(bufa.at[pl.ds(0, ZR)],
                                out_sp.at[pl.ds(off, ZR)])
                return carry

            lax.fori_loop(0, nchunks, zcopy, 0)
            plsc.subcore_barrier()

            def p2chunk(c, carry, h=h):
                base = pl.multiple_of(ebase + c * CE, 8)
                pltpu.sync_copy(src_h.at[pl.ds(base, CE)], srcb)
                pltpu.sync_copy(dst_h.at[pl.ds(base, CE)], dstb)
                pltpu.sync_copy(ex_hbm[t * NH + h].at[pl.ds(base, CE)], abuf)

                nsub = CE // KG

                def fire(g, buf, gi, sm, h=h):
                    for u in range(KG // 16):
                        gi[pl.ds(16 * u, 16)] = srcb[pl.ds(g * KG + 16 * u, 16)]
                    pltpu.async_copy(ht_h[h].at[gi], buf, sm)

                fire(0, bufa, gidx, sem)

                def nrm(j, carry2):
                    d16 = dstb[pl.ds(j * 16, 16)]
                    den = plsc.load_gather(
                        dloc, [lax.shift_right_logical(d16, 7),
                               jnp.bitwise_and(d16, 127)])
                    ex = abuf[pl.ds(j * 16, 16)]
                    abuf[pl.ds(j * 16, 16)] = ex / (den + 1e-16)
                    return carry2

                lax.fori_loop(0, CE // 16, nrm, 0)

                def consume(g, buf, gi, sm):
                    pltpu.make_async_copy(ht_h[h].at[gi], buf, sm).wait()

                    @plsc.parallel_loop(0, KG, unroll=2)
                    def _(r):
                        a = plsc.load_gather(
                            abuf, [lanes * 0 + (g * KG + r)])
                        for v in range(HW // 16):
                            buf[r, pl.ds(16 * v, 16)] = (
                                buf[r, pl.ds(16 * v, 16)] * a
                            )
                    for u in range(KG // 16):
                        sidx[pl.ds(16 * u, 16)] = dstb[pl.ds(g * KG + 16 * u, 16)]
                    pltpu.sync_copy(buf, out_sp.at[sidx], add=True)

                def sub2(gg, carry2, h=h):
                    g0 = gg * 2
                    fire(g0 + 1, bufb, gidxb, ssem)
                    consume(g0, bufa, gidx, sem)
                    fire(g0 + 2, bufa, gidx, sem)
                    consume(g0 + 1, bufb, gidxb, ssem)
                    return carry2

                lax.fori_loop(0, (nsub - 1) // 2, sub2, 0)
                consume(nsub - 1, bufa, gidx, sem)
                return carry

            lax.fori_loop(0, EPT // CE, p2chunk, 0)
            plsc.subcore_barrier()

            def dcopy(j, carry, h=h):
                off = pl.multiple_of(row_start + j * ZR, ZR)
                pltpu.sync_copy(out_sp.at[pl.ds(off, ZR)],
                                out_hbm.at[t, h, pl.ds(off, ZR)])
                return carry

            lax.fori_loop(0, nchunks, dcopy, 0)
            plsc.subcore_barrier()


    @pl.when(cid == 0)
    def _():
        run(0, src0, dst0, as0, ad0, ht0)

    @pl.when(cid == 1)
    def _():
        run(1, src1, dst1, as1, ad1, ht1)


def _sc_edge_conv(src0, dst0, as0, ad0, ht0, src1, dst1, as1, ad1, ht1):
    mesh = plsc.VectorSubcoreMesh(core_axis_name="c", subcore_axis_name="s",
                                  num_cores=2)
    f = pl.kernel(
        _sc_edge_conv_body,
        out_type=(
            jax.ShapeDtypeStruct((2, NHH, NA, HW), jnp.float32),
            [jax.ShapeDtypeStruct((NE,), jnp.float32) for _ in range(2 * NH)],
        ),
        mesh=mesh,
        compiler_params=pltpu.CompilerParams(needs_layout_passes=False),
        scratch_types=[
            pltpu.VMEM((DR, 128), jnp.float32),
            pltpu.VMEM((DR, 128), jnp.float32),
            pltpu.VMEM((DR, 128), jnp.float32),
            pltpu.VMEM((CE,), jnp.float32),
            pltpu.VMEM((CE,), jnp.int32),
            pltpu.VMEM((CE,), jnp.int32),
            pltpu.VMEM((KG,), jnp.int32),
            pltpu.VMEM((KG,), jnp.int32),
            pltpu.VMEM((KG,), jnp.int32),
            pltpu.VMEM((DR,), jnp.int32),
            [pltpu.VMEM_SHARED((DR, 128), jnp.float32) for _ in range(NH)],
            pltpu.VMEM_SHARED((NA, HW), jnp.float32),
            pltpu.SemaphoreType.DMA,
            pltpu.SemaphoreType.DMA,
        ],
    )
    out, _ex = f(src0, dst0, as0, ad0, ht0, src1, dst1, as1, ad1, ht1)
    return out


BN2 = 1000


def _sem_body(o_ref, kw_ref, kb_ref, s_ref):
    i = pl.program_id(1)
    blk = jnp.concatenate([o_ref[0, j] for j in range(NHH)], axis=-1)
    blk = jnp.maximum(blk, 0.0)
    kk = jnp.tanh(
        jnp.dot(blk, kw_ref[...], preferred_element_type=jnp.float32)
        + kb_ref[...]
    )
    part = jnp.sum(kk, axis=0, keepdims=True)[None]

    @pl.when(i == 0)
    def _():
        s_ref[...] = jnp.zeros_like(s_ref)

    s_ref[...] += part


def _sem_sums(out_sc, k_w, k_b):
    return pl.pallas_call(
        _sem_body,
        grid=(2, NA // BN2),
        in_specs=[
            pl.BlockSpec((1, NHH, BN2, HW), lambda t, i: (t, 0, i, 0)),
            pl.BlockSpec((HID, HID), lambda t, i: (0, 0)),
            pl.BlockSpec((1, HID), lambda t, i: (0, 0)),
        ],
        out_specs=pl.BlockSpec((1, 1, HID), lambda t, i: (t, 0, 0)),
        out_shape=jax.ShapeDtypeStruct((2, 1, HID), jnp.float32),
    )(out_sc, k_w, k_b)


def _fuse_body(o_ref, s_ref, q_ref, lw_ref, lb_ref, logits_ref, hrep_ref):
    qv = q_ref[...]
    s0 = jnp.sum(qv[0] * s_ref[0, 0]) / NA
    s1 = jnp.sum(qv[0] * s_ref[1, 0]) / NA
    m = jnp.maximum(s0, s1)
    e0 = jnp.exp(s0 - m)
    e1 = jnp.exp(s1 - m)
    den = e0 + e1
    a0 = e0 / den
    a1 = e1 / den
    b0 = jnp.concatenate([o_ref[0, j] for j in range(NHH)], axis=-1)
    b1 = jnp.concatenate([o_ref[1, j] for j in range(NHH)], axis=-1)
    b0 = jnp.maximum(b0, 0.0)
    b1 = jnp.maximum(b1, 0.0)
    fused = a0 * b0 + a1 * b1
    logits_ref[...] = (
        jnp.dot(fused, lw_ref[...], preferred_element_type=jnp.float32)
        + lb_ref[...]
    )
    hrep_ref[...] = jnp.broadcast_to(fused[:, None, :], (BN2, 2, HID))


def _fuse(out_sc, s2, q, lin_w, lin_b):
    return pl.pallas_call(
        _fuse_body,
        grid=(NA // BN2,),
        in_specs=[
            pl.BlockSpec((2, NHH, BN2, HW), lambda i: (0, 0, i, 0)),
            pl.BlockSpec((2, 1, HID), lambda i: (0, 0, 0)),
            pl.BlockSpec((1, HID), lambda i: (0, 0)),
            pl.BlockSpec((HID, NCLS), lambda i: (0, 0)),
            pl.BlockSpec((1, NCLS), lambda i: (0, 0)),
        ],
        out_specs=[
            pl.BlockSpec((BN2, NCLS), lambda i: (i, 0)),
            pl.BlockSpec((BN2, 2, HID), lambda i: (i, 0, 0)),
        ],
        out_shape=[
            jax.ShapeDtypeStruct((NA, NCLS), jnp.float32),
            jax.ShapeDtypeStruct((NA, 2, HID), jnp.float32),
        ],
    )(out_sc, s2, q, lin_w, lin_b)


def _att_matrix(att_stack):
    att_t = jnp.transpose(att_stack, (1, 2, 0))
    w = att_t[:, :, :, None] * jnp.eye(NH, dtype=att_stack.dtype)[:, None, None, :]
    return w.reshape(HID, att_stack.shape[0] * NH)


def kernel(x_A, x_B, edge_index_ba, edge_index_aa, W_A, b_A, W_B, b_B,
           att_src_ba, att_dst_ba, att_src_aa, att_dst_aa, k_W, k_b, q,
           lin_W, lin_b):
    att_a = jnp.stack(
        [att_dst_ba[0], att_src_aa[0], att_dst_aa[0]]
    )
    att_b = att_src_ba
    watt_a = _att_matrix(att_a)
    watt_b = _att_matrix(att_b)

    xA_p = jnp.pad(x_A, ((0, NAP - NA), (0, 0)))
    xB_p = jnp.pad(x_B, ((0, NAP - NB), (0, 0)))
    outs_a = _project(xA_p, W_A, b_A.reshape(1, HID), watt_a)
    outs_b = _project(xB_p, W_B, b_B.reshape(1, HID), watt_b)
    htA = tuple(outs_a[:NHH])
    htB = tuple(outs_b[:NHH])
    avA = [a.reshape(DR, 128) for a in outs_a[NHH:]]
    avB = [a.reshape(DR, 128) for a in outs_b[NHH:]]
    ad_ba = tuple(avA[0:NH])
    as_aa = tuple(avA[NH:2 * NH])
    ad_aa = tuple(avA[2 * NH:3 * NH])
    as_ba = tuple(avB)

    out_sc = _sc_edge_conv(
        edge_index_ba[0], edge_index_ba[1], as_ba, ad_ba, htB,
        edge_index_aa[0], edge_index_aa[1], as_aa, ad_aa, htA,
    )

    s2 = _sem_sums(out_sc, k_W, k_b.reshape(1, HID))
    logits, hrep = _fuse(out_sc, s2, q.reshape(1, HID), lin_W,
                         lin_b.reshape(1, NCLS))

    alpha = jnp.full((NA, 2), 0.5, jnp.float32)
    return (logits, hrep, alpha)


# speedup vs baseline: 2.8275x; 1.0527x over previous
import functools

import jax
import jax.numpy as jnp
from jax import lax
from jax.experimental import pallas as pl
from jax.experimental.pallas import tpu as pltpu
from jax.experimental.pallas import tpu_sc as plsc

NA = 10000
NB = 10000
NE = 160000
IN_DIM = 256
HID = 512
NH = 4
HD = 128
NCLS = 40

NAP = 10240
NSUB = 16
EPT = NE // NSUB
RPT = 624
CE = 2000
KG = 80
ZR = 16
HW = 128
NSPLIT = HD // HW
NHH = NH * NSPLIT
DR = NAP // 128


BN1 = 512


def _proj_body(x_ref, w_ref, b_ref, watt_ref, *out_refs):
    ka = watt_ref.shape[1]
    ht_refs = out_refs[:NHH]
    av_refs = out_refs[NHH:]
    h = jnp.dot(x_ref[...], w_ref[...], preferred_element_type=jnp.float32)
    h = h + b_ref[...]
    for j in range(NHH):
        ht_refs[j][...] = h[:, j * HW:(j + 1) * HW]
    av = jnp.dot(h, watt_ref[...], preferred_element_type=jnp.float32)
    for j in range(ka):
        av_refs[j][...] = av[:, j:j + 1]


def _project(x, w, b, watt):
    n = x.shape[0]
    ka = watt.shape[1]
    grid = (n // BN1,)
    return pl.pallas_call(
        _proj_body,
        grid=grid,
        in_specs=[
            pl.BlockSpec((BN1, IN_DIM), lambda i: (i, 0)),
            pl.BlockSpec((IN_DIM, HID), lambda i: (0, 0)),
            pl.BlockSpec((1, HID), lambda i: (0, 0)),
            pl.BlockSpec((HID, ka), lambda i: (0, 0)),
        ],
        out_specs=(
            [pl.BlockSpec((BN1, HW), lambda i: (i, 0)) for _ in range(NHH)]
            + [pl.BlockSpec((BN1, 1), lambda i: (i, 0)) for _ in range(ka)]
        ),
        out_shape=(
            [jax.ShapeDtypeStruct((n, HW), jnp.float32) for _ in range(NHH)]
            + [jax.ShapeDtypeStruct((n, 1), jnp.float32) for _ in range(ka)]
        ),
    )(x, w, b, watt)


def _sc_edge_conv_body(
    src0, dst0, as0, ad0, ht0,
    src1, dst1, as1, ad1, ht1,
    out_hbm, ex_hbm,
    bufa, bufb, dloc, abuf, srcb, dstb, gidx, gidxb, sidx, idx80,
    den_sp, out_sp, sem, ssem,
):
    cid = lax.axis_index("c")
    sid = lax.axis_index("s")
    zero16 = jnp.zeros((16,), jnp.float32)
    lanes = lax.iota(jnp.int32, 16)

    def zero_buf(buf):
        def zrow(r, carry):
            for v in range(HW // 16):
                buf[r, pl.ds(16 * v, 16)] = zero16
            return carry

        lax.fori_loop(0, KG, zrow, 0)

    def zero_dloc():
        def zrow(r, carry):
            for v in range(8):
                dloc[r, pl.ds(16 * v, 16)] = zero16
            return carry

        lax.fori_loop(0, DR, zrow, 0)

    def run(t, src_h, dst_h, as_h, ad_h, ht_h):
        ebase = sid * EPT
        row_start = sid * RPT
        nchunks = jnp.where(sid == NSUB - 1, (NA - RPT * (NSUB - 1)) // ZR,
                            RPT // ZR)

        zero_dloc()

        @pl.when(sid == 0)
        def _():
            for h in range(NH):
                pltpu.sync_copy(dloc, den_sp[h])

        for u in range(DR // 16):
            idx80[pl.ds(16 * u, 16)] = lanes + (16 * u)
        plsc.subcore_barrier()

        for h in range(NH):
            pltpu.sync_copy(as_h[h], bufa)
            pltpu.sync_copy(ad_h[h], bufb)
            zero_dloc()

            def p1chunk(c, carry, h=h):
                base = pl.multiple_of(ebase + c * CE, 8)
                pltpu.sync_copy(src_h.at[pl.ds(base, CE)], srcb)
                pltpu.sync_copy(dst_h.at[pl.ds(base, CE)], dstb)

                @plsc.parallel_loop(0, CE // 16, unroll=2)
                def _(j):
                    s16 = srcb[pl.ds(j * 16, 16)]
                    d16 = dstb[pl.ds(j * 16, 16)]
                    sr = lax.shift_right_logical(s16, 7)
                    sc = jnp.bitwise_and(s16, 127)
                    dr = lax.shift_right_logical(d16, 7)
                    dc = jnp.bitwise_and(d16, 127)
                    av = plsc.load_gather(bufa, [sr, sc])
                    bv = plsc.load_gather(bufb, [dr, dc])
                    al = av + bv
                    al = jnp.where(al >= 0.0, al, al * 0.2)
                    ex = jnp.exp(al)
                    abuf[pl.ds(j * 16, 16)] = ex
                    plsc.addupdate_scatter(dloc, [dr, dc], ex)
                pltpu.sync_copy(abuf, ex_hbm[t * NH + h].at[pl.ds(base, CE)])
                return carry

            lax.fori_loop(0, EPT // CE, p1chunk, 0)
            pltpu.sync_copy(dloc, den_sp[h].at[idx80], add=True)

        plsc.subcore_barrier()

        for h in range(NH):
            pltpu.sync_copy(den_sp[h], dloc)
            zero_buf(bufa)

            def zcopy(j, carry):
                off = pl.multiple_of(row_start + j * ZR, ZR)
                pltpu.sync_copy(bufa.at[pl.ds(0, ZR)],
                                out_sp.at[pl.ds(off, ZR)])
                return carry

            lax.fori_loop(0, nchunks, zcopy, 0)
            plsc.subcore_barrier()

            def p2chunk(c, carry, h=h):
                base = pl.multiple_of(ebase + c * CE, 8)
                pltpu.sync_copy(src_h.at[pl.ds(base, CE)], srcb)
                pltpu.sync_copy(dst_h.at[pl.ds(base, CE)], dstb)
                pltpu.sync_copy(ex_hbm[t * NH + h].at[pl.ds(base, CE)], abuf)

                nsub = CE // KG

                def fire(g, buf, gi, sm, h=h):
                    for u in range(KG // 16):
                        gi[pl.ds(16 * u, 16)] = srcb[pl.ds(g * KG + 16 * u, 16)]
                    pltpu.async_copy(ht_h[h].at[gi], buf, sm)

                fire(0, bufa, gidx, sem)

                @plsc.parallel_loop(0, CE // 16, unroll=2)
                def _(j):
                    d16 = dstb[pl.ds(j * 16, 16)]
                    den = plsc.load_gather(
                        dloc, [lax.shift_right_logical(d16, 7),
                               jnp.bitwise_and(d16, 127)])
                    ex = abuf[pl.ds(j * 16, 16)]
                    abuf[pl.ds(j * 16, 16)] = ex / (den + 1e-16)

                def consume(g, buf, gi, sm):
                    pltpu.make_async_copy(ht_h[h].at[gi], buf, sm).wait()

                    @plsc.parallel_loop(0, KG, unroll=2)
                    def _(r):
                        a = plsc.load_gather(
                            abuf, [lanes * 0 + (g * KG + r)])
                        for v in range(HW // 16):
                            buf[r, pl.ds(16 * v, 16)] = (
                                buf[r, pl.ds(16 * v, 16)] * a
                            )
                    for u in range(KG // 16):
                        sidx[pl.ds(16 * u, 16)] = dstb[pl.ds(g * KG + 16 * u, 16)]
                    pltpu.sync_copy(buf, out_sp.at[sidx], add=True)

                def sub2(gg, carry2, h=h):
                    g0 = gg * 2
                    fire(g0 + 1, bufb, gidxb, ssem)
                    consume(g0, bufa, gidx, sem)
                    fire(g0 + 2, bufa, gidx, sem)
                    consume(g0 + 1, bufb, gidxb, ssem)
                    return carry2

                lax.fori_loop(0, (nsub - 1) // 2, sub2, 0)
                consume(nsub - 1, bufa, gidx, sem)
                return carry

            lax.fori_loop(0, EPT // CE, p2chunk, 0)
            plsc.subcore_barrier()

            def dcopy(j, carry, h=h):
                off = pl.multiple_of(row_start + j * ZR, ZR)
                pltpu.sync_copy(out_sp.at[pl.ds(off, ZR)],
                                out_hbm.at[t, h, pl.ds(off, ZR)])
                return carry

            lax.fori_loop(0, nchunks, dcopy, 0)
            plsc.subcore_barrier()


    @pl.when(cid == 0)
    def _():
        run(0, src0, dst0, as0, ad0, ht0)

    @pl.when(cid == 1)
    def _():
        run(1, src1, dst1, as1, ad1, ht1)


def _sc_edge_conv(src0, dst0, as0, ad0, ht0, src1, dst1, as1, ad1, ht1):
    mesh = plsc.VectorSubcoreMesh(core_axis_name="c", subcore_axis_name="s",
                                  num_cores=2)
    f = pl.kernel(
        _sc_edge_conv_body,
        out_type=(
            jax.ShapeDtypeStruct((2, NHH, NA, HW), jnp.float32),
            [jax.ShapeDtypeStruct((NE,), jnp.float32) for _ in range(2 * NH)],
        ),
        mesh=mesh,
        compiler_params=pltpu.CompilerParams(needs_layout_passes=False),
        scratch_types=[
            pltpu.VMEM((DR, 128), jnp.float32),
            pltpu.VMEM((DR, 128), jnp.float32),
            pltpu.VMEM((DR, 128), jnp.float32),
            pltpu.VMEM((CE,), jnp.float32),
            pltpu.VMEM((CE,), jnp.int32),
            pltpu.VMEM((CE,), jnp.int32),
            pltpu.VMEM((KG,), jnp.int32),
            pltpu.VMEM((KG,), jnp.int32),
            pltpu.VMEM((KG,), jnp.int32),
            pltpu.VMEM((DR,), jnp.int32),
            [pltpu.VMEM_SHARED((DR, 128), jnp.float32) for _ in range(NH)],
            pltpu.VMEM_SHARED((NA, HW), jnp.float32),
            pltpu.SemaphoreType.DMA,
            pltpu.SemaphoreType.DMA,
        ],
    )
    out, _ex = f(src0, dst0, as0, ad0, ht0, src1, dst1, as1, ad1, ht1)
    return out


BN2 = 1000


def _sem_body(o_ref, kw_ref, kb_ref, s_ref):
    i = pl.program_id(1)
    blk = jnp.concatenate([o_ref[0, j] for j in range(NHH)], axis=-1)
    blk = jnp.maximum(blk, 0.0)
    kk = jnp.tanh(
        jnp.dot(blk, kw_ref[...], preferred_element_type=jnp.float32)
        + kb_ref[...]
    )
    part = jnp.sum(kk, axis=0, keepdims=True)[None]

    @pl.when(i == 0)
    def _():
        s_ref[...] = jnp.zeros_like(s_ref)

    s_ref[...] += part


def _sem_sums(out_sc, k_w, k_b):
    return pl.pallas_call(
        _sem_body,
        grid=(2, NA // BN2),
        in_specs=[
            pl.BlockSpec((1, NHH, BN2, HW), lambda t, i: (t, 0, i, 0)),
            pl.BlockSpec((HID, HID), lambda t, i: (0, 0)),
            pl.BlockSpec((1, HID), lambda t, i: (0, 0)),
        ],
        out_specs=pl.BlockSpec((1, 1, HID), lambda t, i: (t, 0, 0)),
        out_shape=jax.ShapeDtypeStruct((2, 1, HID), jnp.float32),
    )(out_sc, k_w, k_b)


def _fuse_body(o_ref, s_ref, q_ref, lw_ref, lb_ref, logits_ref, hrep_ref):
    qv = q_ref[...]
    s0 = jnp.sum(qv[0] * s_ref[0, 0]) / NA
    s1 = jnp.sum(qv[0] * s_ref[1, 0]) / NA
    m = jnp.maximum(s0, s1)
    e0 = jnp.exp(s0 - m)
    e1 = jnp.exp(s1 - m)
    den = e0 + e1
    a0 = e0 / den
    a1 = e1 / den
    b0 = jnp.concatenate([o_ref[0, j] for j in range(NHH)], axis=-1)
    b1 = jnp.concatenate([o_ref[1, j] for j in range(NHH)], axis=-1)
    b0 = jnp.maximum(b0, 0.0)
    b1 = jnp.maximum(b1, 0.0)
    fused = a0 * b0 + a1 * b1
    logits_ref[...] = (
        jnp.dot(fused, lw_ref[...], preferred_element_type=jnp.float32)
        + lb_ref[...]
    )
    hrep_ref[...] = jnp.broadcast_to(fused[:, None, :], (BN2, 2, HID))


def _fuse(out_sc, s2, q, lin_w, lin_b):
    return pl.pallas_call(
        _fuse_body,
        grid=(NA // BN2,),
        in_specs=[
            pl.BlockSpec((2, NHH, BN2, HW), lambda i: (0, 0, i, 0)),
            pl.BlockSpec((2, 1, HID), lambda i: (0, 0, 0)),
            pl.BlockSpec((1, HID), lambda i: (0, 0)),
            pl.BlockSpec((HID, NCLS), lambda i: (0, 0)),
            pl.BlockSpec((1, NCLS), lambda i: (0, 0)),
        ],
        out_specs=[
            pl.BlockSpec((BN2, NCLS), lambda i: (i, 0)),
            pl.BlockSpec((BN2, 2, HID), lambda i: (i, 0, 0)),
        ],
        out_shape=[
            jax.ShapeDtypeStruct((NA, NCLS), jnp.float32),
            jax.ShapeDtypeStruct((NA, 2, HID), jnp.float32),
        ],
    )(out_sc, s2, q, lin_w, lin_b)


def _att_matrix(att_stack):
    att_t = jnp.transpose(att_stack, (1, 2, 0))
    w = att_t[:, :, :, None] * jnp.eye(NH, dtype=att_stack.dtype)[:, None, None, :]
    return w.reshape(HID, att_stack.shape[0] * NH)


def kernel(x_A, x_B, edge_index_ba, edge_index_aa, W_A, b_A, W_B, b_B,
           att_src_ba, att_dst_ba, att_src_aa, att_dst_aa, k_W, k_b, q,
           lin_W, lin_b):
    att_a = jnp.stack(
        [att_dst_ba[0], att_src_aa[0], att_dst_aa[0]]
    )
    att_b = att_src_ba
    watt_a = _att_matrix(att_a)
    watt_b = _att_matrix(att_b)

    xA_p = jnp.pad(x_A, ((0, NAP - NA), (0, 0)))
    xB_p = jnp.pad(x_B, ((0, NAP - NB), (0, 0)))
    outs_a = _project(xA_p, W_A, b_A.reshape(1, HID), watt_a)
    outs_b = _project(xB_p, W_B, b_B.reshape(1, HID), watt_b)
    htA = tuple(outs_a[:NHH])
    htB = tuple(outs_b[:NHH])
    avA = [a.reshape(DR, 128) for a in outs_a[NHH:]]
    avB = [a.reshape(DR, 128) for a in outs_b[NHH:]]
    ad_ba = tuple(avA[0:NH])
    as_aa = tuple(avA[NH:2 * NH])
    ad_aa = tuple(avA[2 * NH:3 * NH])
    as_ba = tuple(avB)

    out_sc = _sc_edge_conv(
        edge_index_ba[0], edge_index_ba[1], as_ba, ad_ba, htB,
        edge_index_aa[0], edge_index_aa[1], as_aa, ad_aa, htA,
    )

    s2 = _sem_sums(out_sc, k_W, k_b.reshape(1, HID))
    logits, hrep = _fuse(out_sc, s2, q.reshape(1, HID), lin_W,
                         lin_b.reshape(1, NCLS))

    alpha = jnp.full((NA, 2), 0.5, jnp.float32)
    return (logits, hrep, alpha)
